# baseline (speedup 1.0000x reference)
import functools

import jax
import jax.numpy as jnp
from jax import lax
from jax.experimental import pallas as pl
from jax.experimental.pallas import tpu as pltpu
from jax.experimental.pallas import tpu_sc as plsc

N = 50000
D = 128
E = 625000
SW = 16
NSLICE = D // SW
C = 1024
NCH = 39
NBUF = 3
NSUB = 16
E_W = NCH * C
E_PAD = NSUB * E_W
N_PAD = 50048
RS = N_PAD // NSUB
ZB_ROWS = 184
CZB = 2000

BR = 2000
NB = N // BR


def _stats_body(v_ref, c_ref, out_ref):
    i = pl.program_id(0)

    @pl.when(i == 0)
    def _():
        out_ref[...] = jnp.zeros_like(out_ref)

    v = v_ref[...]
    c = c_ref[...]
    blk = jnp.concatenate(
        [
            jnp.sum(v, axis=0, keepdims=True),
            jnp.sum(v * v, axis=0, keepdims=True),
            jnp.sum(c, axis=0, keepdims=True),
            jnp.sum(c * c, axis=0, keepdims=True),
            jnp.zeros((4, D), jnp.float32),
        ],
        axis=0,
    )
    out_ref[...] += blk


def _apply_body(v_ref, c_ref, st_ref, gn_ref, bn_ref, gc_ref, bc_ref,
                wrn_ref, wrc_ref, wtn_ref, wtc_ref,
                zc_ref, zn_ref, rn_ref, rc_ref):
    st = st_ref[...]
    m_n = st[0:1, :] / N
    var_n = st[1:2, :] / N - m_n * m_n
    m_c = st[2:3, :] / N
    var_c = st[3:4, :] / N - m_c * m_c
    sc_n = lax.rsqrt(var_n + 1e-5) * gn_ref[...]
    sc_c = lax.rsqrt(var_c + 1e-5) * gc_ref[...]
    xn = (v_ref[...] - m_n) * sc_n + bn_ref[...]
    xc = (c_ref[...] - m_c) * sc_c + bc_ref[...]
    zc_ref[...] = jnp.dot(xc, wrn_ref[...], preferred_element_type=jnp.float32)
    zn_ref[...] = jnp.dot(xn, wrc_ref[...], preferred_element_type=jnp.float32)
    rn_ref[...] = jnp.dot(xn, wtn_ref[...], preferred_element_type=jnp.float32)
    rc_ref[...] = jnp.dot(xc, wtc_ref[...], preferred_element_type=jnp.float32)


def _final_body(sn_ref, sc_ref, cn_ref, cc_ref, rn_ref, rc_ref,
                brn_ref, brc_ref, on_ref, oc_ref):
    aggn = sn_ref[...] / jnp.maximum(cn_ref[...], 1.0)
    aggc = sc_ref[...] / jnp.maximum(cc_ref[...], 1.0)
    on_ref[...] = jnp.maximum(aggn + brn_ref[...] + rn_ref[...], 0.0)
    oc_ref[...] = jnp.maximum(aggc + brc_ref[...] + rc_ref[...], 0.0)


def _sc_body(zt_c, zt_n, srcp, dstp, ewp, valp,
             sn_out, sc_out, cn_out, cc_out,
             acc, cnt, rows, gidx, sidx, ewv, vv, sx, vx, zb, czb,
             semg, semm, semsc):
    cid = lax.axis_index("c")
    w = lax.axis_index("s")

    zeros16 = jnp.zeros((16,), jnp.float32)

    @plsc.parallel_loop(0, ZB_ROWS, 1, unroll=4)
    def _z1(i):
        zb[i, pl.ds(0, 16)] = zeros16

    @plsc.parallel_loop(0, CZB // 16, 1, unroll=4)
    def _z2(i):
        czb[pl.ds(i * 16, 16)] = zeros16

    def run_dir(table, gq, sq, out_hbm, cnt_hbm):
        @pl.when(w == 0)
        def _():
            for t in range(N // CZB):
                pltpu.sync_copy(czb, cnt.at[pl.ds(t * CZB, CZB)])

        @pl.loop(0, NSLICE)
        def _slice(s):
            is0 = s == 0

            def issue_meta(k, b):
                base = w * E_W + k * C
                pltpu.async_copy(gq.at[pl.ds(base, C)], gidx[b], semm[b])
                pltpu.async_copy(sq.at[pl.ds(base, C)], sidx[b], semm[b])
                pltpu.async_copy(ewp.at[pl.ds(base, C)], ewv[b], semm[b])

                @pl.when(is0)
                def _():
                    pltpu.async_copy(valp.at[pl.ds(base, C)], vv[b], semm[b])

            def wait_meta(b):
                for _ in range(3):
                    pltpu.make_async_copy(
                        gq.at[pl.ds(0, C)], gidx[b], semm[b]).wait()

                @pl.when(is0)
                def _():
                    pltpu.make_async_copy(
                        gq.at[pl.ds(0, C)], gidx[b], semm[b]).wait()

            def bias_and_gather(b):
                @plsc.parallel_loop(0, C // 16, 1, unroll=4)
                def _bias(i):
                    gidx[b][pl.ds(i * 16, 16)] = (
                        gidx[b][pl.ds(i * 16, 16)] * 8 + s)

                pltpu.async_copy(table.at[gidx[b]], rows[b], semg[b])

            def wait_scatter(b):
                pltpu.make_async_copy(
                    table.at[pl.ds(0, C)], rows[b], semsc[b]).wait()

                @pl.when(is0)
                def _():
                    pltpu.make_async_copy(
                        ewp.at[pl.ds(0, C)], vx[b], semsc[b]).wait()

            for t in range(RS // ZB_ROWS):
                pltpu.sync_copy(
                    zb, acc.at[pl.ds(w * RS + t * ZB_ROWS, ZB_ROWS)])
            plsc.subcore_barrier()

            issue_meta(0, 0)
            issue_meta(1, 1)
            wait_meta(0)
            bias_and_gather(0)

            @pl.loop(0, NCH // NBUF)
            def _chunk(g):
                for a in range(NBUF):
                    k = NBUF * g + a
                    nxt = (a + 1) % NBUF
                    prv = (a + 2) % NBUF
                    pltpu.make_async_copy(
                        table.at[pl.ds(0, C)], rows[a], semg[a]).wait()

                    @pl.when(k + 2 < NCH)
                    def _():
                        issue_meta(k + 2, prv)

                    @pl.when(k >= 2)
                    def _():
                        wait_scatter(nxt)

                    @pl.when(k + 1 < NCH)
                    def _():
                        wait_meta(nxt)
                        bias_and_gather(nxt)

                    @plsc.parallel_loop(0, C // 16, 1, unroll=2)
                    def _grp(gg):
                        ew16 = ewv[a][pl.ds(gg * 16, 16)]
                        for j in range(16):
                            r = gg * 16 + j
                            e = ew16[j]
                            rows[a][r, pl.ds(0, 16)] = (
                                rows[a][r, pl.ds(0, 16)] * e)

                    @plsc.parallel_loop(0, C // 16, 1, unroll=4)
                    def _cpx(i):
                        sx[a][pl.ds(i * 16, 16)] = sidx[a][pl.ds(i * 16, 16)]

                    pltpu.async_copy(rows[a], acc.at[sx[a]], semsc[a],
                                     add=True)

                    @pl.when(is0)
                    def _():
                        @plsc.parallel_loop(0, C // 16, 1, unroll=4)
                        def _cpv(i):
                            vx[a][pl.ds(i * 16, 16)] = (
                                vv[a][pl.ds(i * 16, 16)])

                        pltpu.async_copy(vx[a], cnt.at[sx[a]], semsc[a],
                                         add=True)

            wait_scatter((NCH - 2) % NBUF)
            wait_scatter((NCH - 1) % NBUF)
            plsc.subcore_barrier()
            pltpu.sync_copy(
                acc.at[pl.ds(w * RS, RS)],
                out_hbm.at[pl.ds(w * RS, RS), pl.ds(s * SW, SW)])

        plsc.subcore_barrier()

        @pl.when(w == 0)
        def _():
            pltpu.sync_copy(cnt, cnt_hbm)

    @pl.when(cid == 0)
    def _():
        run_dir(zt_c, srcp, dstp, sn_out, cn_out)

    @pl.when(cid == 1)
    def _():
        run_dir(zt_n, dstp, srcp, sc_out, cc_out)


@functools.cache
def _make_sc_call():
  return pl.kernel(
    _sc_body,
    out_type=(
        jax.ShapeDtypeStruct((N_PAD, D), jnp.float32),
        jax.ShapeDtypeStruct((N_PAD, D), jnp.float32),
        jax.ShapeDtypeStruct((N,), jnp.float32),
        jax.ShapeDtypeStruct((N,), jnp.float32),
    ),
    mesh=plsc.VectorSubcoreMesh(core_axis_name="c", subcore_axis_name="s"),
    compiler_params=pltpu.CompilerParams(use_tc_tiling_on_sc=False),
    scratch_types=[
        pltpu.VMEM_SHARED((N_PAD, SW), jnp.float32),
        pltpu.VMEM_SHARED((N,), jnp.float32),
        tuple(pltpu.VMEM((C, SW), jnp.float32) for _ in range(NBUF)),
        tuple(pltpu.VMEM((C,), jnp.int32) for _ in range(NBUF)),
        tuple(pltpu.VMEM((C,), jnp.int32) for _ in range(NBUF)),
        tuple(pltpu.VMEM((C,), jnp.float32) for _ in range(NBUF)),
        tuple(pltpu.VMEM((C,), jnp.float32) for _ in range(NBUF)),
        tuple(pltpu.VMEM((C,), jnp.int32) for _ in range(NBUF)),
        tuple(pltpu.VMEM((C,), jnp.float32) for _ in range(NBUF)),
        pltpu.VMEM((ZB_ROWS, SW), jnp.float32),
        pltpu.VMEM((CZB,), jnp.float32),
        tuple(pltpu.SemaphoreType.DMA for _ in range(NBUF)),
        tuple(pltpu.SemaphoreType.DMA for _ in range(NBUF)),
        tuple(pltpu.SemaphoreType.DMA for _ in range(NBUF)),
    ],
  )


def kernel(var_feats, cstr_feats, edge_index, edge_attr, gamma_n, beta_n,
           gamma_c, beta_c, W_rel_n, b_rel_n, W_root_n, W_rel_c, b_rel_c,
           W_root_c):
    f32 = jnp.float32
    stats = pl.pallas_call(
        _stats_body,
        grid=(NB,),
        in_specs=[
            pl.BlockSpec((BR, D), lambda i: (i, 0)),
            pl.BlockSpec((BR, D), lambda i: (i, 0)),
        ],
        out_specs=pl.BlockSpec((8, D), lambda i: (0, 0)),
        out_shape=jax.ShapeDtypeStruct((8, D), f32),
    )(var_feats, cstr_feats)

    row = lambda a: a.reshape(1, D)
    full = lambda shape: pl.BlockSpec(shape, lambda i: (0, 0))
    blk = pl.BlockSpec((BR, D), lambda i: (i, 0))
    z_c, z_n, r_n, r_c = pl.pallas_call(
        _apply_body,
        grid=(NB,),
        in_specs=[blk, blk, full((8, D)), full((1, D)), full((1, D)),
                  full((1, D)), full((1, D)), full((D, D)), full((D, D)),
                  full((D, D)), full((D, D))],
        out_specs=[blk, blk, blk, blk],
        out_shape=[jax.ShapeDtypeStruct((N, D), f32)] * 4,
    )(var_feats, cstr_feats, stats, row(gamma_n), row(beta_n), row(gamma_c),
      row(beta_c), W_rel_n, W_rel_c, W_root_n, W_root_c)

    npad = E_PAD - E
    pad_idx = (jnp.arange(npad, dtype=jnp.int32) * 97) % N
    src_p = jnp.concatenate([edge_index[0], pad_idx])
    dst_p = jnp.concatenate([edge_index[1], pad_idx])
    ew_p = jnp.concatenate([edge_attr, jnp.zeros((npad,), f32)])
    val_p = jnp.concatenate([jnp.ones((E,), f32), jnp.zeros((npad,), f32)])

    s_n, s_c, cnt_n, cnt_c = _make_sc_call()(
        z_c.reshape(NSLICE * N, SW), z_n.reshape(NSLICE * N, SW),
        src_p, dst_p, ew_p, val_p)

    cblk = pl.BlockSpec((BR, 1), lambda i: (i, 0))
    out_node, out_cstr = pl.pallas_call(
        _final_body,
        grid=(NB,),
        in_specs=[blk, blk, cblk, cblk, blk, blk, full((1, D)),
                  full((1, D))],
        out_specs=[blk, blk],
        out_shape=[jax.ShapeDtypeStruct((N, D), f32)] * 2,
    )(s_n, s_c, cnt_n.reshape(N, 1), cnt_c.reshape(N, 1), r_n, r_c,
      row(b_rel_n), row(b_rel_c))
    return (out_node, out_cstr)


# speedup vs baseline: 1.0004x; 1.0004x over previous
import functools

import jax
import jax.numpy as jnp
from jax import lax
from jax.experimental import pallas as pl
from jax.experimental.pallas import tpu as pltpu
from jax.experimental.pallas import tpu_sc as plsc

N = 50000
D = 128
E = 625000
SW = 16
NSLICE = D // SW
C = 1024
NCH = 39
NBUF = 3
NSUB = 16
E_W = NCH * C
E_PAD = NSUB * E_W
N_PAD = 50048
RS = N_PAD // NSUB
ZB_ROWS = 184
CZB = 2000

BR = 2000
NB = N // BR


def _stats_body(v_ref, c_ref, out_ref):
    i = pl.program_id(0)

    @pl.when(i == 0)
    def _():
        out_ref[...] = jnp.zeros_like(out_ref)

    v = v_ref[...]
    c = c_ref[...]
    blk = jnp.concatenate(
        [
            jnp.sum(v, axis=0, keepdims=True),
            jnp.sum(v * v, axis=0, keepdims=True),
            jnp.sum(c, axis=0, keepdims=True),
            jnp.sum(c * c, axis=0, keepdims=True),
            jnp.zeros((4, D), jnp.float32),
        ],
        axis=0,
    )
    out_ref[...] += blk


def _apply_body(v_ref, c_ref, st_ref, gn_ref, bn_ref, gc_ref, bc_ref,
                wrn_ref, wrc_ref, wtn_ref, wtc_ref,
                zc_ref, zn_ref, rn_ref, rc_ref):
    st = st_ref[...]
    m_n = st[0:1, :] / N
    var_n = st[1:2, :] / N - m_n * m_n
    m_c = st[2:3, :] / N
    var_c = st[3:4, :] / N - m_c * m_c
    sc_n = lax.rsqrt(var_n + 1e-5) * gn_ref[...]
    sc_c = lax.rsqrt(var_c + 1e-5) * gc_ref[...]
    xn = (v_ref[...] - m_n) * sc_n + bn_ref[...]
    xc = (c_ref[...] - m_c) * sc_c + bc_ref[...]
    zc_ref[...] = jnp.dot(xc, wrn_ref[...], preferred_element_type=jnp.float32)
    zn_ref[...] = jnp.dot(xn, wrc_ref[...], preferred_element_type=jnp.float32)
    rn_ref[...] = jnp.dot(xn, wtn_ref[...], preferred_element_type=jnp.float32)
    rc_ref[...] = jnp.dot(xc, wtc_ref[...], preferred_element_type=jnp.float32)


def _final_body(sn_ref, sc_ref, cn_ref, cc_ref, rn_ref, rc_ref,
                brn_ref, brc_ref, on_ref, oc_ref):
    aggn = sn_ref[...] / jnp.maximum(cn_ref[...], 1.0)
    aggc = sc_ref[...] / jnp.maximum(cc_ref[...], 1.0)
    on_ref[...] = jnp.maximum(aggn + brn_ref[...] + rn_ref[...], 0.0)
    oc_ref[...] = jnp.maximum(aggc + brc_ref[...] + rc_ref[...], 0.0)


def _sc_body(zt_c, zt_n, srcp, dstp, ewp, valp,
             sn_out, sc_out, cn_out, cc_out,
             acc, cnt, rows, gidx, sidx, ewv, vv, zb, czb,
             semg, semm, semsc):
    cid = lax.axis_index("c")
    w = lax.axis_index("s")

    zeros16 = jnp.zeros((16,), jnp.float32)

    @plsc.parallel_loop(0, ZB_ROWS, 1, unroll=4)
    def _z1(i):
        zb[i, pl.ds(0, 16)] = zeros16

    @plsc.parallel_loop(0, CZB // 16, 1, unroll=4)
    def _z2(i):
        czb[pl.ds(i * 16, 16)] = zeros16

    def run_dir(table, gq, sq, out_hbm, cnt_hbm):
        @pl.when(w == 0)
        def _():
            for t in range(N // CZB):
                pltpu.sync_copy(czb, cnt.at[pl.ds(t * CZB, CZB)])

        @pl.loop(0, NSLICE)
        def _slice(s):
            is0 = s == 0

            def issue_meta(k, b):
                base = w * E_W + k * C
                pltpu.async_copy(gq.at[pl.ds(base, C)], gidx[b], semm[b])
                pltpu.async_copy(sq.at[pl.ds(base, C)], sidx[b], semm[b])
                pltpu.async_copy(ewp.at[pl.ds(base, C)], ewv[b], semm[b])

                @pl.when(is0)
                def _():
                    pltpu.async_copy(valp.at[pl.ds(base, C)], vv[b], semm[b])

            def wait_meta(b):
                for _ in range(3):
                    pltpu.make_async_copy(
                        gq.at[pl.ds(0, C)], gidx[b], semm[b]).wait()

                @pl.when(is0)
                def _():
                    pltpu.make_async_copy(
                        gq.at[pl.ds(0, C)], gidx[b], semm[b]).wait()

            def bias_and_gather(b):
                @plsc.parallel_loop(0, C // 16, 1, unroll=4)
                def _bias(i):
                    gidx[b][pl.ds(i * 16, 16)] = (
                        gidx[b][pl.ds(i * 16, 16)] * 8 + s)

                pltpu.async_copy(table.at[gidx[b]], rows[b], semg[b])

            def wait_scatter(b):
                pltpu.make_async_copy(
                    table.at[pl.ds(0, C)], rows[b], semsc[b]).wait()

                @pl.when(is0)
                def _():
                    pltpu.make_async_copy(
                        ewp.at[pl.ds(0, C)], vv[b], semsc[b]).wait()

            for t in range(RS // ZB_ROWS):
                pltpu.sync_copy(
                    zb, acc.at[pl.ds(w * RS + t * ZB_ROWS, ZB_ROWS)])
            plsc.subcore_barrier()

            issue_meta(0, 0)
            issue_meta(1, 1)
            wait_meta(0)
            bias_and_gather(0)

            @pl.loop(0, NCH // NBUF)
            def _chunk(g):
                for a in range(NBUF):
                    k = NBUF * g + a
                    nxt = (a + 1) % NBUF
                    prv = (a + 2) % NBUF
                    pltpu.make_async_copy(
                        table.at[pl.ds(0, C)], rows[a], semg[a]).wait()

                    @pl.when(k >= 1)
                    def _():
                        wait_scatter(prv)

                    @pl.when(k + 2 < NCH)
                    def _():
                        issue_meta(k + 2, prv)

                    @pl.when(k + 1 < NCH)
                    def _():
                        wait_meta(nxt)
                        bias_and_gather(nxt)

                    @plsc.parallel_loop(0, C // 16, 1, unroll=1)
                    def _grp(gg):
                        ew16 = ewv[a][pl.ds(gg * 16, 16)]
                        for j in range(16):
                            r = gg * 16 + j
                            e = ew16[j]
                            rows[a][r, pl.ds(0, 16)] = (
                                rows[a][r, pl.ds(0, 16)] * e)

                    pltpu.async_copy(rows[a], acc.at[sidx[a]], semsc[a],
                                     add=True)

                    @pl.when(is0)
                    def _():
                        pltpu.async_copy(vv[a], cnt.at[sidx[a]], semsc[a],
                                         add=True)

            wait_scatter((NCH - 1) % NBUF)
            plsc.subcore_barrier()
            pltpu.sync_copy(
                acc.at[pl.ds(w * RS, RS)],
                out_hbm.at[pl.ds(w * RS, RS), pl.ds(s * SW, SW)])

        plsc.subcore_barrier()

        @pl.when(w == 0)
        def _():
            pltpu.sync_copy(cnt, cnt_hbm)

    @pl.when(cid == 0)
    def _():
        run_dir(zt_c, srcp, dstp, sn_out, cn_out)

    @pl.when(cid == 1)
    def _():
        run_dir(zt_n, dstp, srcp, sc_out, cc_out)


@functools.cache
def _make_sc_call():
  return pl.kernel(
    _sc_body,
    out_type=(
        jax.ShapeDtypeStruct((N_PAD, D), jnp.float32),
        jax.ShapeDtypeStruct((N_PAD, D), jnp.float32),
        jax.ShapeDtypeStruct((N,), jnp.float32),
        jax.ShapeDtypeStruct((N,), jnp.float32),
    ),
    mesh=plsc.VectorSubcoreMesh(core_axis_name="c", subcore_axis_name="s"),
    compiler_params=pltpu.CompilerParams(use_tc_tiling_on_sc=False),
    scratch_types=[
        pltpu.VMEM_SHARED((N_PAD, SW), jnp.float32),
        pltpu.VMEM_SHARED((N,), jnp.float32),
        tuple(pltpu.VMEM((C, SW), jnp.float32) for _ in range(NBUF)),
        tuple(pltpu.VMEM((C,), jnp.int32) for _ in range(NBUF)),
        tuple(pltpu.VMEM((C,), jnp.int32) for _ in range(NBUF)),
        tuple(pltpu.VMEM((C,), jnp.float32) for _ in range(NBUF)),
        tuple(pltpu.VMEM((C,), jnp.float32) for _ in range(NBUF)),
        pltpu.VMEM((ZB_ROWS, SW), jnp.float32),
        pltpu.VMEM((CZB,), jnp.float32),
        tuple(pltpu.SemaphoreType.DMA for _ in range(NBUF)),
        tuple(pltpu.SemaphoreType.DMA for _ in range(NBUF)),
        tuple(pltpu.SemaphoreType.DMA for _ in range(NBUF)),
    ],
  )


def kernel(var_feats, cstr_feats, edge_index, edge_attr, gamma_n, beta_n,
           gamma_c, beta_c, W_rel_n, b_rel_n, W_root_n, W_rel_c, b_rel_c,
           W_root_c):
    f32 = jnp.float32
    stats = pl.pallas_call(
        _stats_body,
        grid=(NB,),
        in_specs=[
            pl.BlockSpec((BR, D), lambda i: (i, 0)),
            pl.BlockSpec((BR, D), lambda i: (i, 0)),
        ],
        out_specs=pl.BlockSpec((8, D), lambda i: (0, 0)),
        out_shape=jax.ShapeDtypeStruct((8, D), f32),
    )(var_feats, cstr_feats)

    row = lambda a: a.reshape(1, D)
    full = lambda shape: pl.BlockSpec(shape, lambda i: (0, 0))
    blk = pl.BlockSpec((BR, D), lambda i: (i, 0))
    z_c, z_n, r_n, r_c = pl.pallas_call(
        _apply_body,
        grid=(NB,),
        in_specs=[blk, blk, full((8, D)), full((1, D)), full((1, D)),
                  full((1, D)), full((1, D)), full((D, D)), full((D, D)),
                  full((D, D)), full((D, D))],
        out_specs=[blk, blk, blk, blk],
        out_shape=[jax.ShapeDtypeStruct((N, D), f32)] * 4,
    )(var_feats, cstr_feats, stats, row(gamma_n), row(beta_n), row(gamma_c),
      row(beta_c), W_rel_n, W_rel_c, W_root_n, W_root_c)

    npad = E_PAD - E
    pad_idx = (jnp.arange(npad, dtype=jnp.int32) * 97) % N
    src_p = jnp.concatenate([edge_index[0], pad_idx])
    dst_p = jnp.concatenate([edge_index[1], pad_idx])
    ew_p = jnp.concatenate([edge_attr, jnp.zeros((npad,), f32)])
    val_p = jnp.concatenate([jnp.ones((E,), f32), jnp.zeros((npad,), f32)])

    s_n, s_c, cnt_n, cnt_c = _make_sc_call()(
        z_c.reshape(NSLICE * N, SW), z_n.reshape(NSLICE * N, SW),
        src_p, dst_p, ew_p, val_p)

    cblk = pl.BlockSpec((BR, 1), lambda i: (i, 0))
    out_node, out_cstr = pl.pallas_call(
        _final_body,
        grid=(NB,),
        in_specs=[blk, blk, cblk, cblk, blk, blk, full((1, D)),
                  full((1, D))],
        out_specs=[blk, blk],
        out_shape=[jax.ShapeDtypeStruct((N, D), f32)] * 2,
    )(s_n, s_c, cnt_n.reshape(N, 1), cnt_c.reshape(N, 1), r_n, r_c,
      row(b_rel_n), row(b_rel_c))
    return (out_node, out_cstr)


# speedup vs baseline: 1.1368x; 1.1364x over previous
import functools

import jax
import jax.numpy as jnp
from jax import lax
from jax.experimental import pallas as pl
from jax.experimental.pallas import tpu as pltpu
from jax.experimental.pallas import tpu_sc as plsc

N = 50000
D = 128
E = 625000
SW = 16
NSLICE = D // SW
C = 1024
NCH = 39
NBUF = 3
NSUB = 16
E_W = NCH * C
E_PAD = NSUB * E_W
N_PAD = 50048
RS = N_PAD // NSUB
ZB_ROWS = 184
CZB = 2000

BR = 2000
NB = N // BR


def _stats_body(v_ref, c_ref, out_ref):
    i = pl.program_id(0)

    @pl.when(i == 0)
    def _():
        out_ref[...] = jnp.zeros_like(out_ref)

    v = v_ref[...]
    c = c_ref[...]
    blk = jnp.concatenate(
        [
            jnp.sum(v, axis=0, keepdims=True),
            jnp.sum(v * v, axis=0, keepdims=True),
            jnp.sum(c, axis=0, keepdims=True),
            jnp.sum(c * c, axis=0, keepdims=True),
            jnp.zeros((4, D), jnp.float32),
        ],
        axis=0,
    )
    out_ref[...] += blk


def _apply_body(v_ref, c_ref, st_ref, gn_ref, bn_ref, gc_ref, bc_ref,
                wrn_ref, wrc_ref, wtn_ref, wtc_ref,
                zc_ref, zn_ref, rn_ref, rc_ref):
    st = st_ref[...]
    m_n = st[0:1, :] / N
    var_n = st[1:2, :] / N - m_n * m_n
    m_c = st[2:3, :] / N
    var_c = st[3:4, :] / N - m_c * m_c
    sc_n = lax.rsqrt(var_n + 1e-5) * gn_ref[...]
    sc_c = lax.rsqrt(var_c + 1e-5) * gc_ref[...]
    xn = (v_ref[...] - m_n) * sc_n + bn_ref[...]
    xc = (c_ref[...] - m_c) * sc_c + bc_ref[...]
    zc_ref[...] = jnp.dot(xc, wrn_ref[...], preferred_element_type=jnp.float32)
    zn_ref[...] = jnp.dot(xn, wrc_ref[...], preferred_element_type=jnp.float32)
    rn_ref[...] = jnp.dot(xn, wtn_ref[...], preferred_element_type=jnp.float32)
    rc_ref[...] = jnp.dot(xc, wtc_ref[...], preferred_element_type=jnp.float32)


def _final_body(sn_ref, sc_ref, cn_ref, cc_ref, rn_ref, rc_ref,
                brn_ref, brc_ref, on_ref, oc_ref):
    aggn = sn_ref[...] / jnp.maximum(cn_ref[...], 1.0)
    aggc = sc_ref[...] / jnp.maximum(cc_ref[...], 1.0)
    on_ref[...] = jnp.maximum(aggn + brn_ref[...] + rn_ref[...], 0.0)
    oc_ref[...] = jnp.maximum(aggc + brc_ref[...] + rc_ref[...], 0.0)


def _sc_body(zt_c, zt_n, srcp, dstp, ewp, valp,
             sn_out, sc_out, cn_out, cc_out,
             acc, cnt, rows, gidx, sidx, ewv, vv, zb, czb, stg,
             semg, semm, semsc, semw):
    cid = lax.axis_index("c")
    w = lax.axis_index("s")

    zeros16 = jnp.zeros((16,), jnp.float32)

    @plsc.parallel_loop(0, ZB_ROWS, 1, unroll=4)
    def _z1(i):
        zb[i, pl.ds(0, 16)] = zeros16

    @plsc.parallel_loop(0, CZB // 16, 1, unroll=4)
    def _z2(i):
        czb[pl.ds(i * 16, 16)] = zeros16

    def run_dir(table, gq, sq, out_hbm, cnt_hbm):
        @pl.when(w == 0)
        def _():
            for t in range(N // CZB):
                pltpu.sync_copy(czb, cnt.at[pl.ds(t * CZB, CZB)])

        for t in range(RS // ZB_ROWS):
            pltpu.sync_copy(zb, acc.at[pl.ds(w * RS + t * ZB_ROWS, ZB_ROWS)])

        @pl.loop(0, NSLICE)
        def _slice(s):
            is0 = s == 0

            def issue_meta(k, b):
                base = w * E_W + k * C
                pltpu.async_copy(gq.at[pl.ds(base, C)], gidx[b], semm[b])
                pltpu.async_copy(sq.at[pl.ds(base, C)], sidx[b], semm[b])
                pltpu.async_copy(ewp.at[pl.ds(base, C)], ewv[b], semm[b])

                @pl.when(is0)
                def _():
                    pltpu.async_copy(valp.at[pl.ds(base, C)], vv[b], semm[b])

            def wait_meta(b):
                for _ in range(3):
                    pltpu.make_async_copy(
                        gq.at[pl.ds(0, C)], gidx[b], semm[b]).wait()

                @pl.when(is0)
                def _():
                    pltpu.make_async_copy(
                        gq.at[pl.ds(0, C)], gidx[b], semm[b]).wait()

            def bias_and_gather(b):
                @plsc.parallel_loop(0, C // 16, 1, unroll=4)
                def _bias(i):
                    gidx[b][pl.ds(i * 16, 16)] = (
                        gidx[b][pl.ds(i * 16, 16)] * 8 + s)

                pltpu.async_copy(table.at[gidx[b]], rows[b], semg[b])

            def wait_scatter(b):
                pltpu.make_async_copy(
                    table.at[pl.ds(0, C)], rows[b], semsc[b]).wait()

                @pl.when(is0)
                def _():
                    pltpu.make_async_copy(
                        ewp.at[pl.ds(0, C)], vv[b], semsc[b]).wait()

            plsc.subcore_barrier()

            issue_meta(0, 0)
            issue_meta(1, 1)
            wait_meta(0)
            bias_and_gather(0)

            @pl.loop(0, NCH // NBUF)
            def _chunk(g):
                for a in range(NBUF):
                    k = NBUF * g + a
                    nxt = (a + 1) % NBUF
                    prv = (a + 2) % NBUF
                    pltpu.make_async_copy(
                        table.at[pl.ds(0, C)], rows[a], semg[a]).wait()

                    @pl.when(k >= 1)
                    def _():
                        wait_scatter(prv)

                    @pl.when(k + 2 < NCH)
                    def _():
                        issue_meta(k + 2, prv)

                    @pl.when(k + 1 < NCH)
                    def _():
                        wait_meta(nxt)
                        bias_and_gather(nxt)

                    @plsc.parallel_loop(0, C // 16, 1, unroll=1)
                    def _grp(gg):
                        ew16 = ewv[a][pl.ds(gg * 16, 16)]
                        for j in range(16):
                            r = gg * 16 + j
                            e = ew16[j]
                            rows[a][r, pl.ds(0, 16)] = (
                                rows[a][r, pl.ds(0, 16)] * e)

                    pltpu.async_copy(rows[a], acc.at[sidx[a]], semsc[a],
                                     add=True)

                    @pl.when(is0)
                    def _():
                        pltpu.async_copy(vv[a], cnt.at[sidx[a]], semsc[a],
                                         add=True)

            wait_scatter((NCH - 1) % NBUF)
            plsc.subcore_barrier()
            for t in range(RS // ZB_ROWS):
                tt = t % 2
                r0 = w * RS + t * ZB_ROWS
                wr_wait = pltpu.make_async_copy(
                    stg[tt],
                    out_hbm.at[pl.ds(r0, ZB_ROWS), pl.ds(s * SW, SW)],
                    semw[tt])
                if t >= 2:
                    wr_wait.wait()
                else:
                    @pl.when(s >= 1)
                    def _():
                        wr_wait.wait()
                pltpu.sync_copy(acc.at[pl.ds(r0, ZB_ROWS)], stg[tt])
                pltpu.async_copy(
                    stg[tt],
                    out_hbm.at[pl.ds(r0, ZB_ROWS), pl.ds(s * SW, SW)],
                    semw[tt])
                pltpu.sync_copy(zb, acc.at[pl.ds(r0, ZB_ROWS)])

        for tt in range(2):
            pltpu.make_async_copy(
                stg[tt],
                out_hbm.at[pl.ds(0, ZB_ROWS), pl.ds(0, SW)],
                semw[tt]).wait()
        plsc.subcore_barrier()

        @pl.when(w == 0)
        def _():
            pltpu.sync_copy(cnt, cnt_hbm)

    @pl.when(cid == 0)
    def _():
        run_dir(zt_c, srcp, dstp, sn_out, cn_out)

    @pl.when(cid == 1)
    def _():
        run_dir(zt_n, dstp, srcp, sc_out, cc_out)


@functools.cache
def _make_sc_call():
  return pl.kernel(
    _sc_body,
    out_type=(
        jax.ShapeDtypeStruct((N_PAD, D), jnp.float32),
        jax.ShapeDtypeStruct((N_PAD, D), jnp.float32),
        jax.ShapeDtypeStruct((N,), jnp.float32),
        jax.ShapeDtypeStruct((N,), jnp.float32),
    ),
    mesh=plsc.VectorSubcoreMesh(core_axis_name="c", subcore_axis_name="s"),
    compiler_params=pltpu.CompilerParams(use_tc_tiling_on_sc=False),
    scratch_types=[
        pltpu.VMEM_SHARED((N_PAD, SW), jnp.float32),
        pltpu.VMEM_SHARED((N,), jnp.float32),
        tuple(pltpu.VMEM((C, SW), jnp.float32) for _ in range(NBUF)),
        tuple(pltpu.VMEM((C,), jnp.int32) for _ in range(NBUF)),
        tuple(pltpu.VMEM((C,), jnp.int32) for _ in range(NBUF)),
        tuple(pltpu.VMEM((C,), jnp.float32) for _ in range(NBUF)),
        tuple(pltpu.VMEM((C,), jnp.float32) for _ in range(NBUF)),
        pltpu.VMEM((ZB_ROWS, SW), jnp.float32),
        pltpu.VMEM((CZB,), jnp.float32),
        (pltpu.VMEM((ZB_ROWS, SW), jnp.float32),
         pltpu.VMEM((ZB_ROWS, SW), jnp.float32)),
        tuple(pltpu.SemaphoreType.DMA for _ in range(NBUF)),
        tuple(pltpu.SemaphoreType.DMA for _ in range(NBUF)),
        tuple(pltpu.SemaphoreType.DMA for _ in range(NBUF)),
        (pltpu.SemaphoreType.DMA, pltpu.SemaphoreType.DMA),
    ],
  )


def kernel(var_feats, cstr_feats, edge_index, edge_attr, gamma_n, beta_n,
           gamma_c, beta_c, W_rel_n, b_rel_n, W_root_n, W_rel_c, b_rel_c,
           W_root_c):
    f32 = jnp.float32
    stats = pl.pallas_call(
        _stats_body,
        grid=(NB,),
        in_specs=[
            pl.BlockSpec((BR, D), lambda i: (i, 0)),
            pl.BlockSpec((BR, D), lambda i: (i, 0)),
        ],
        out_specs=pl.BlockSpec((8, D), lambda i: (0, 0)),
        out_shape=jax.ShapeDtypeStruct((8, D), f32),
    )(var_feats, cstr_feats)

    row = lambda a: a.reshape(1, D)
    full = lambda shape: pl.BlockSpec(shape, lambda i: (0, 0))
    blk = pl.BlockSpec((BR, D), lambda i: (i, 0))
    z_c, z_n, r_n, r_c = pl.pallas_call(
        _apply_body,
        grid=(NB,),
        in_specs=[blk, blk, full((8, D)), full((1, D)), full((1, D)),
                  full((1, D)), full((1, D)), full((D, D)), full((D, D)),
                  full((D, D)), full((D, D))],
        out_specs=[blk, blk, blk, blk],
        out_shape=[jax.ShapeDtypeStruct((N, D), f32)] * 4,
    )(var_feats, cstr_feats, stats, row(gamma_n), row(beta_n), row(gamma_c),
      row(beta_c), W_rel_n, W_rel_c, W_root_n, W_root_c)

    npad = E_PAD - E
    pad_idx = (jnp.arange(npad, dtype=jnp.int32) * 97) % N
    src_p = jnp.concatenate([edge_index[0], pad_idx])
    dst_p = jnp.concatenate([edge_index[1], pad_idx])
    ew_p = jnp.concatenate([edge_attr, jnp.zeros((npad,), f32)])
    val_p = jnp.concatenate([jnp.ones((E,), f32), jnp.zeros((npad,), f32)])

    s_n, s_c, cnt_n, cnt_c = _make_sc_call()(
        z_c.reshape(NSLICE * N, SW), z_n.reshape(NSLICE * N, SW),
        src_p, dst_p, ew_p, val_p)

    cblk = pl.BlockSpec((BR, 1), lambda i: (i, 0))
    out_node, out_cstr = pl.pallas_call(
        _final_body,
        grid=(NB,),
        in_specs=[blk, blk, cblk, cblk, blk, blk, full((1, D)),
                  full((1, D))],
        out_specs=[blk, blk],
        out_shape=[jax.ShapeDtypeStruct((N, D), f32)] * 2,
    )(s_n, s_c, cnt_n.reshape(N, 1), cnt_c.reshape(N, 1), r_n, r_c,
      row(b_rel_n), row(b_rel_c))
    return (out_node, out_cstr)
